# baseline (speedup 1.0000x reference)
import functools

import jax
import jax.numpy as jnp
from jax import lax
from jax.experimental import pallas as pl
from jax.experimental.pallas import tpu as pltpu
from jax.experimental.pallas import tpu_sc as plsc

_R, _C = 128, 100000
_W = 8192
_NB = pl.cdiv(_C, _W)


def _make_gumbels():
    u = jax.random.uniform(jax.random.key(42), (_R, _C), dtype=jnp.float32)
    return -jnp.log(-jnp.log(u + 1e-10) + 1e-10)


_GUMBELS = _make_gumbels()


_ZW = 12800
_ZW_LAST = 12832
_ZOFF_LAST = _C - _ZW_LAST


def _zeros_sc_kernel(out_hbm, zbuf, sem):
    w = lax.axis_index("s") * 2 + lax.axis_index("c")

    z16 = jnp.zeros((16,), jnp.float32)
    for r in range(8):

        def _fill(i, _, r=r):
            zbuf[r, pl.ds(i * 16, 16)] = z16
            return _

        lax.fori_loop(0, _ZW_LAST // 16, _fill, 0)

    hi = w // 16
    band = w % 16

    def _dma(chunk):
        off = _ZW * chunk if chunk < 7 else _ZOFF_LAST
        width = _ZW if chunk < 7 else _ZW_LAST
        return pltpu.make_async_copy(
            zbuf.at[:, pl.ds(0, width)],
            out_hbm.at[pl.ds(pl.multiple_of(8 * band, 8), 8),
                       pl.ds(off, width)],
            sem,
        )

    for k in range(4):
        for h in (0, 1):

            @pl.when(hi == h)
            def _(k=k, h=h):
                _dma(2 * k + h).start()

    for k in range(4):
        for h in (0, 1):

            @pl.when(hi == h)
            def _(k=k, h=h):
                _dma(2 * k + h).wait()


def _make_zeros_sc():
    return pl.kernel(
        _zeros_sc_kernel,
        out_type=jax.ShapeDtypeStruct((_R, _C), jnp.float32),
        mesh=plsc.VectorSubcoreMesh(core_axis_name="c", subcore_axis_name="s"),
        scratch_types=[
            pltpu.VMEM((8, _ZW_LAST), jnp.float32),
            pltpu.SemaphoreType.DMA,
        ],
        compiler_params=pltpu.CompilerParams(use_tc_tiling_on_sc=True),
    )


def _argmax_kernel(x_ref, g_ref, idx_ref, val_ref):
    j = pl.program_id(0)
    cols = j * _W + jax.lax.broadcasted_iota(jnp.int32, (_R, _W), 1)
    x = x_ref[...] + g_ref[...]
    x = jnp.where(cols < _C, x, -jnp.inf)

    @pl.when(j == 0)
    def _init():
        val_ref[...] = jnp.full((_R, 1), -jnp.inf, jnp.float32)
        idx_ref[...] = jnp.zeros((_R, 1), jnp.int32)

    bmax = jnp.max(x, axis=1, keepdims=True)
    cand = jnp.where(x == bmax, cols, 2**31 - 1)
    bidx = jnp.min(cand, axis=1, keepdims=True)
    better = bmax > val_ref[...]
    val_ref[...] = jnp.where(better, bmax, val_ref[...])
    idx_ref[...] = jnp.where(better, bidx, idx_ref[...])


_G = _R // 8


def _scatter_kernel(idx_smem, idx3_vmem, zeros_any, out_any, stage, sem):
    idx3 = idx3_vmem[...]
    lane = jax.lax.broadcasted_iota(jnp.int32, (_G, 8, 128), 2)
    for rsub in range(8):
        c0 = (idx3[:, rsub : rsub + 1, :] // 128) * 128
        stage[rsub] = jnp.where(idx3 == c0 + lane, 1.0, 0.0).astype(
            jnp.float32
        )

    def _tile_copy(g, rsub):
        c0 = pl.multiple_of((idx_smem[8 * g + rsub] // 128) * 128, 128)
        return pltpu.make_async_copy(
            stage.at[rsub, g],
            out_any.at[pl.ds(8 * g, 8), pl.ds(c0, 128)],
            sem,
        )

    for g in range(_G):
        for rsub in range(8):
            _tile_copy(g, rsub).start()
    for g in range(_G):
        for rsub in range(8):
            _tile_copy(g, rsub).wait()


def kernel(logits):
    g = _GUMBELS
    zeros = _make_zeros_sc()()
    idx, _ = pl.pallas_call(
        _argmax_kernel,
        grid=(_NB,),
        in_specs=[
            pl.BlockSpec((_R, _W), lambda j: (0, j)),
            pl.BlockSpec((_R, _W), lambda j: (0, j)),
        ],
        out_specs=[
            pl.BlockSpec((_R, 1), lambda j: (0, 0)),
            pl.BlockSpec((_R, 1), lambda j: (0, 0)),
        ],
        out_shape=[
            jax.ShapeDtypeStruct((_R, 1), jnp.int32),
            jax.ShapeDtypeStruct((_R, 1), jnp.float32),
        ],
    )(logits, g)

    idx_flat = idx.reshape(_R)
    idx3 = idx.reshape(_G, 8, 1)
    out = pl.pallas_call(
        _scatter_kernel,
        in_specs=[
            pl.BlockSpec(memory_space=pltpu.SMEM),
            pl.BlockSpec(memory_space=pltpu.VMEM),
            pl.BlockSpec(memory_space=pltpu.MemorySpace.HBM),
        ],
        out_specs=pl.BlockSpec(memory_space=pltpu.MemorySpace.HBM),
        out_shape=jax.ShapeDtypeStruct((_R, _C), jnp.float32),
        scratch_shapes=[
            pltpu.VMEM((8, _G, 8, 128), jnp.float32),
            pltpu.SemaphoreType.DMA,
        ],
        input_output_aliases={2: 0},
    )(idx_flat, idx3, zeros)
    return out


# speedup vs baseline: 2.1397x; 2.1397x over previous
import jax
import jax.numpy as jnp
from jax import lax
from jax.experimental import pallas as pl
from jax.experimental.pallas import tpu as pltpu
from jax.experimental.pallas import tpu_sc as plsc

_R, _C = 128, 100000
_WT = 8192
_NBT = pl.cdiv(_C, _WT)


def _make_gumbels_t():
    u = jax.random.uniform(jax.random.key(42), (_R, _C), dtype=jnp.float32)
    g = -jnp.log(-jnp.log(u + 1e-10) + 1e-10)
    return g.T


_GUMBELS_T = _make_gumbels_t()


_ZROWS = 800
_ZTASKS = _C // _ZROWS


def _zeros_sc_kernel(out_hbm, zbuf, sem):
    w = lax.axis_index("s") * 2 + lax.axis_index("c")

    z16 = jnp.zeros((16,), jnp.float32)

    def _fill(i, c):
        for k in range(8):
            zbuf[i, pl.ds(k * 16, 16)] = z16
        return c

    lax.fori_loop(0, _ZROWS, _fill, 0)

    def _dma(t):
        return pltpu.make_async_copy(
            zbuf,
            out_hbm.at[pl.ds(pl.multiple_of(_ZROWS * t, 8), _ZROWS), :],
            sem,
        )

    for k in range(4):

        @pl.when(w + 32 * k < _ZTASKS)
        def _(k=k):
            _dma(w + 32 * k).start()

    for k in range(4):

        @pl.when(w + 32 * k < _ZTASKS)
        def _(k=k):
            _dma(w + 32 * k).wait()


def _make_zeros_sc():
    return pl.kernel(
        _zeros_sc_kernel,
        out_type=jax.ShapeDtypeStruct((_C, _R), jnp.float32),
        mesh=plsc.VectorSubcoreMesh(core_axis_name="c", subcore_axis_name="s"),
        scratch_types=[
            pltpu.VMEM((_ZROWS, _R), jnp.float32),
            pltpu.SemaphoreType.DMA,
        ],
        compiler_params=pltpu.CompilerParams(use_tc_tiling_on_sc=True),
    )


def _argmax_kernel(x_ref, g_ref, idx_ref, val_ref):
    j = pl.program_id(0)
    rows = j * _WT + jax.lax.broadcasted_iota(jnp.int32, (_WT, _R), 0)
    x = x_ref[...] + g_ref[...]
    x = jnp.where(rows < _C, x, -jnp.inf)

    @pl.when(j == 0)
    def _init():
        val_ref[...] = jnp.full((1, _R), -jnp.inf, jnp.float32)
        idx_ref[...] = jnp.zeros((1, _R), jnp.int32)

    bmax = jnp.max(x, axis=0, keepdims=True)
    cand = jnp.where(x == bmax, rows, 2**31 - 1)
    bidx = jnp.min(cand, axis=0, keepdims=True)
    better = bmax > val_ref[...]
    val_ref[...] = jnp.where(better, bmax, val_ref[...])
    idx_ref[...] = jnp.where(better, bidx, idx_ref[...])


def _scatter_kernel(idx_smem, idx_a, idx_b, zeros_hbm, out_hbm, stage, sem):
    s_iota = jax.lax.broadcasted_iota(jnp.int32, (_R, 8, _R), 1)
    c0b = (idx_b[...] // 8) * 8
    stage[...] = jnp.where(idx_a[...] == c0b + s_iota, 1.0, 0.0).astype(
        jnp.float32
    )

    def _dma(l):
        c0 = pl.multiple_of((idx_smem[l] // 8) * 8, 8)
        return pltpu.make_async_copy(
            stage.at[l],
            out_hbm.at[pl.ds(c0, 8), :],
            sem,
        )

    for l in range(_R):
        _dma(l).start()
    for l in range(_R):
        _dma(l).wait()


def kernel(logits):
    lt = logits.T
    zeros = _make_zeros_sc()()

    idxv, _ = pl.pallas_call(
        _argmax_kernel,
        grid=(_NBT,),
        in_specs=[
            pl.BlockSpec((_WT, _R), lambda j: (j, 0)),
            pl.BlockSpec((_WT, _R), lambda j: (j, 0)),
        ],
        out_specs=[
            pl.BlockSpec((1, _R), lambda j: (0, 0)),
            pl.BlockSpec((1, _R), lambda j: (0, 0)),
        ],
        out_shape=[
            jax.ShapeDtypeStruct((1, _R), jnp.int32),
            jax.ShapeDtypeStruct((1, _R), jnp.float32),
        ],
    )(lt, _GUMBELS_T)

    idx_flat = idxv.reshape(_R)
    idx_a = idxv.reshape(1, 1, _R)
    idx_b = idxv.reshape(_R, 1, 1)
    out_t = pl.pallas_call(
        _scatter_kernel,
        in_specs=[
            pl.BlockSpec(memory_space=pltpu.SMEM),
            pl.BlockSpec(memory_space=pltpu.VMEM),
            pl.BlockSpec(memory_space=pltpu.VMEM),
            pl.BlockSpec(memory_space=pltpu.MemorySpace.HBM),
        ],
        out_specs=pl.BlockSpec(memory_space=pltpu.MemorySpace.HBM),
        out_shape=jax.ShapeDtypeStruct((_C, _R), jnp.float32),
        scratch_shapes=[
            pltpu.VMEM((_R, 8, _R), jnp.float32),
            pltpu.SemaphoreType.DMA,
        ],
        input_output_aliases={3: 0},
    )(idx_flat, idx_a, idx_b, zeros)
    return out_t.T
